# baseline (speedup 1.0000x reference)
import functools

import jax
import jax.numpy as jnp
from jax import lax
from jax.experimental import pallas as pl
from jax.experimental.pallas import tpu as pltpu
from jax.experimental.pallas import tpu_sc as plsc

_NC = 2
_NS = 16
_NW = _NC * _NS
_C = 256
_NBUF = 4


@functools.partial(jax.jit, static_argnums=(2, 3))
def _gather_rows(table, idx, B, D):
  BPW = B // _NW
  NCH = BPW // _C
  G = NCH // _NBUF
  mesh = plsc.VectorSubcoreMesh(
      core_axis_name="c", subcore_axis_name="s",
      num_cores=_NC, num_subcores=_NS)

  @functools.partial(
      pl.kernel,
      out_type=jax.ShapeDtypeStruct((B, D), jnp.float32),
      mesh=mesh,
      scratch_types=[
          pltpu.VMEM((BPW,), jnp.int32),
          pltpu.VMEM((_NBUF, _C, D), jnp.float32),
          pltpu.SemaphoreType.DMA((_NBUF,)),
          pltpu.SemaphoreType.DMA((_NBUF,)),
      ],
      compiler_params=pltpu.CompilerParams(use_tc_tiling_on_sc=False),
  )
  def gather_kernel(table_hbm, idx_hbm, out_hbm, idx_v, rows_v, gsem, osem):
    wid = lax.axis_index("s") * _NC + lax.axis_index("c")
    base = wid * BPW
    pltpu.sync_copy(idx_hbm.at[pl.ds(base, BPW)], idx_v)

    def gd(j, b):
      return pltpu.make_async_copy(
          table_hbm.at[idx_v.at[pl.ds(j * _C, _C)]], rows_v.at[b], gsem.at[b])

    def od(j, b):
      return pltpu.make_async_copy(
          rows_v.at[b], out_hbm.at[pl.ds(base + j * _C, _C)], osem.at[b])

    for b in range(_NBUF):
      gd(b, b).start()
    for b in range(_NBUF):
      gd(b, b).wait()
      od(b, b).start()

    @pl.loop(1, G)
    def _(g):
      j0 = g * _NBUF
      for b in range(_NBUF):
        od(j0 - _NBUF + b, b).wait()
        gd(j0 + b, b).start()
      for b in range(_NBUF):
        gd(j0 + b, b).wait()
        od(j0 + b, b).start()

    for b in range(_NBUF):
      od((G - 1) * _NBUF + b, b).wait()

  return gather_kernel(table, idx)


def kernel(token_ids, token_table, op_table, var_table, const_table,
           struct_table, special_table):
  batch, seq = token_ids.shape
  half = token_table.shape[1]
  d = 2 * half
  cat = jnp.concatenate(
      [op_table, var_table, const_table, struct_table, special_table], axis=0)
  fused = jnp.concatenate([token_table, cat], axis=1)
  idx = token_ids.reshape(-1).astype(jnp.int32)
  out = _gather_rows(fused, idx, idx.shape[0], d)
  return out.reshape(batch, seq, d)


# speedup vs baseline: 1.0593x; 1.0593x over previous
import functools

import jax
import jax.numpy as jnp
from jax import lax
from jax.experimental import pallas as pl
from jax.experimental.pallas import tpu as pltpu
from jax.experimental.pallas import tpu_sc as plsc

_NC = 2
_NS = 16
_NW = _NC * _NS
_C = 256
_NBUF = 4


@functools.partial(jax.jit, static_argnums=(2, 3))
def _gather_rows(table, idx, B, D):
  BPW = B // _NW
  NCH = BPW // _C
  G = NCH // _NBUF
  mesh = plsc.VectorSubcoreMesh(
      core_axis_name="c", subcore_axis_name="s",
      num_cores=_NC, num_subcores=_NS)

  @functools.partial(
      pl.kernel,
      out_type=jax.ShapeDtypeStruct((B, D), jnp.float32),
      mesh=mesh,
      scratch_types=[
          pltpu.VMEM((BPW,), jnp.int32),
          pltpu.VMEM((_NBUF, _C, D), jnp.float32),
          pltpu.SemaphoreType.DMA((_NBUF,)),
          pltpu.SemaphoreType.DMA((_NBUF,)),
      ],
      compiler_params=pltpu.CompilerParams(use_tc_tiling_on_sc=False),
  )
  def gather_kernel(table_hbm, idx_hbm, out_hbm, idx_v, rows_v, gsem, osem):
    wid = lax.axis_index("s") * _NC + lax.axis_index("c")
    base = wid * BPW
    pltpu.sync_copy(idx_hbm.at[pl.ds(base, BPW)], idx_v)

    def gd(j, b):
      return pltpu.make_async_copy(
          table_hbm.at[idx_v.at[pl.ds(j * _C, _C)]], rows_v.at[b], gsem.at[b])

    def od(j, b):
      return pltpu.make_async_copy(
          rows_v.at[b], out_hbm.at[pl.ds(base + j * _C, _C)], osem.at[b])

    for b in range(_NBUF):
      gd(b, b).start()
    for b in range(_NBUF):
      gd(b, b).wait()
      od(b, b).start()

    @pl.loop(1, G)
    def _(g):
      j0 = g * _NBUF
      for b in range(_NBUF):
        od(j0 - _NBUF + b, b).wait()
        gd(j0 + b, b).start()
      for b in range(_NBUF):
        gd(j0 + b, b).wait()
        od(j0 + b, b).start()

    for b in range(_NBUF):
      od((G - 1) * _NBUF + b, b).wait()

  return gather_kernel(table, idx)


_BLK = 2000


@jax.jit
def _build_fused(token_table, op_table, var_table, const_table, struct_table,
                 special_table):
  vocab, half = token_table.shape

  def body(tok_ref, op_ref, var_ref, const_ref, struct_ref, spec_prev_ref,
           spec_cur_ref, out_ref):
    i = pl.program_id(0)
    tail = spec_prev_ref[_BLK - 40:_BLK, :]
    first = jnp.concatenate(
        [op_ref[...], var_ref[...], const_ref[...], struct_ref[...]], axis=0)
    top40 = jnp.where(i == 0, first, tail)
    cat_block = jnp.concatenate([top40, spec_cur_ref[:_BLK - 40, :]], axis=0)
    out_ref[...] = jnp.concatenate([tok_ref[...], cat_block], axis=1)

  grid = vocab // _BLK
  return pl.pallas_call(
      body,
      grid=(grid,),
      in_specs=[
          pl.BlockSpec((_BLK, half), lambda i: (i, 0)),
          pl.BlockSpec((10, half), lambda i: (0, 0)),
          pl.BlockSpec((10, half), lambda i: (0, 0)),
          pl.BlockSpec((10, half), lambda i: (0, 0)),
          pl.BlockSpec((10, half), lambda i: (0, 0)),
          pl.BlockSpec((_BLK, half), lambda i: (jnp.maximum(i - 1, 0), 0)),
          pl.BlockSpec((_BLK, half), lambda i: (i, 0)),
      ],
      out_specs=pl.BlockSpec((_BLK, 2 * half), lambda i: (i, 0)),
      out_shape=jax.ShapeDtypeStruct((vocab, 2 * half), jnp.float32),
  )(token_table, op_table, var_table, const_table, struct_table,
    special_table, special_table)


def kernel(token_ids, token_table, op_table, var_table, const_table,
           struct_table, special_table):
  batch, seq = token_ids.shape
  half = token_table.shape[1]
  d = 2 * half
  fused = _build_fused(token_table, op_table, var_table, const_table,
                       struct_table, special_table)
  idx = token_ids.reshape(-1).astype(jnp.int32)
  out = _gather_rows(fused, idx, idx.shape[0], d)
  return out.reshape(batch, seq, d)
